# baseline (speedup 1.0000x reference)
import jax
import jax.numpy as jnp
from jax.experimental import pallas as pl
from jax.experimental.pallas import tpu as pltpu

_N = 2048
_F = 32
_K = 8
_IT = 256


def _colsum(x):
    acc = jnp.sum(x.reshape(_N // 8, 8, x.shape[-1]), axis=0)
    r4 = acc[:4, :] + acc[4:, :]
    r2 = r4[:2, :] + r4[2:, :]
    return r2[0:1, :] + r2[1:2, :]


def _normalize(x):
    sq = x * x
    acc = sq[:, 0:8] + sq[:, 8:16]
    acc = acc + sq[:, 16:24]
    acc = acc + sq[:, 24:32]
    r4 = acc[:, 0:4] + acc[:, 4:8]
    r2 = r4[:, 0:2] + r4[:, 2:4]
    nsq = r2[:, 0:1] + r2[:, 1:2]
    return x / jnp.sqrt(nsq)


def _softmax_mean_cols(st):
    cmax = jnp.max(st, axis=0, keepdims=True)
    e = jnp.exp(st - cmax)
    den = _colsum(e)
    p = e / den
    return _colsum(p) * jnp.float32(1.0 / _N)


def _fusion_kernel(ref_rgb_ref, ref_flow_ref, sup_rgb_ref, sup_flow_ref,
                   out_ref_rgb_ref, out_sup_rgb_ref, m_intra_ref, m_inter_ref):
    rn = _normalize(ref_flow_ref[0])
    sn = _normalize(sup_flow_ref[0])

    for t in range(_N // _IT):
        lo, hi = t * _IT, (t + 1) * _IT
        st_intra = jax.lax.dot_general(
            rn, rn[lo:hi, :], (((1,), (1,)), ((), ())),
            preferred_element_type=jnp.float32)
        m_intra_ref[0:1, lo:hi] = _softmax_mean_cols(st_intra)
        st_inter = jax.lax.dot_general(
            rn, sn[lo:hi, :], (((1,), (1,)), ((), ())),
            preferred_element_type=jnp.float32)
        m_inter_ref[0:1, lo:hi] = _softmax_mean_cols(st_inter)

    iota = jax.lax.broadcasted_iota(jnp.int32, (1, _N), 1)

    m = m_intra_ref[0, :][None, :]
    for k in range(_K):
        v = jnp.min(m)
        idx = jnp.min(jnp.where(m == v, iota, _N))
        out_ref_rgb_ref[0, k, :] = ref_rgb_ref[0, pl.ds(idx, 1), :][0]
        m = jnp.where(iota == idx, jnp.inf, m)

    m = m_inter_ref[0, :][None, :]
    for k in range(_K):
        v = jnp.max(m)
        idx = jnp.min(jnp.where(m == v, iota, _N))
        out_sup_rgb_ref[0, k, :] = sup_rgb_ref[0, pl.ds(idx, 1), :][0]
        m = jnp.where(iota == idx, -jnp.inf, m)


def _run(ref_rgb_feat, ref_flow_feat, sup_rgb_feat, sup_flow_feat):
    B = ref_rgb_feat.shape[0]
    feat_spec = pl.BlockSpec((1, _N, _F), lambda b: (b, 0, 0))
    out_spec = pl.BlockSpec((1, _K, _F), lambda b: (b, 0, 0))
    out_rgb, out_sup = pl.pallas_call(
        _fusion_kernel,
        grid=(B,),
        in_specs=[feat_spec, feat_spec, feat_spec, feat_spec],
        out_specs=[out_spec, out_spec],
        out_shape=[
            jax.ShapeDtypeStruct((B, _K, _F), jnp.float32),
            jax.ShapeDtypeStruct((B, _K, _F), jnp.float32),
        ],
        scratch_shapes=[
            pltpu.VMEM((1, _N), jnp.float32),
            pltpu.VMEM((1, _N), jnp.float32),
        ],
    )(ref_rgb_feat, ref_flow_feat, sup_rgb_feat, sup_flow_feat)
    return (out_rgb, out_sup)


@jax.jit
def kernel(ref_rgb_feat, ref_flow_feat, sup_rgb_feat, sup_flow_feat):
    devs = jax.devices()
    B = ref_rgb_feat.shape[0]
    if len(devs) >= 2 and B % 2 == 0:
        import numpy as _np
        mesh = jax.sharding.Mesh(_np.array(devs[:2]), ("x",))
        P = jax.sharding.PartitionSpec
        fn = jax.shard_map(_run, mesh=mesh,
                           in_specs=(P("x"),) * 4,
                           out_specs=(P("x"), P("x")),
                           check_vma=False)
        return fn(ref_rgb_feat, ref_flow_feat, sup_rgb_feat, sup_flow_feat)
    return _run(ref_rgb_feat, ref_flow_feat, sup_rgb_feat, sup_flow_feat)


# speedup vs baseline: 1.6141x; 1.6141x over previous
import jax
import jax.numpy as jnp
from jax.experimental import pallas as pl
from jax.experimental.pallas import tpu as pltpu

_N = 2048
_F = 32
_K = 8
_IT = 256


def _colsum(x):
    acc = jnp.sum(x.reshape(_N // 8, 8, x.shape[-1]), axis=0)
    return _halve8(acc)


def _normalize(x):
    sq = x * x
    acc = sq[:, 0:8] + sq[:, 8:16]
    acc = acc + sq[:, 16:24]
    acc = acc + sq[:, 24:32]
    r4 = acc[:, 0:4] + acc[:, 4:8]
    r2 = r4[:, 0:2] + r4[:, 2:4]
    nsq = r2[:, 0:1] + r2[:, 1:2]
    return x / jnp.sqrt(nsq)


def _halve8(acc):
    r4 = acc[:4, :] + acc[4:, :]
    r2 = r4[:2, :] + r4[2:, :]
    return r2[0:1, :] + r2[1:2, :]


def _softmax_mean_cols(st, e_ref):
    cmax = jnp.max(st, axis=0, keepdims=True)
    e = jnp.exp(st - cmax)
    e_ref[:, :] = e
    den = _colsum(e)

    def body(c, acc):
        return acc + e_ref[pl.ds(8 * c, 8), :] / den

    acc = jax.lax.fori_loop(1, _N // 8, body, e_ref[0:8, :] / den)
    return _halve8(acc) * jnp.float32(1.0 / _N)


def _fusion_kernel(ref_rgb_ref, ref_flow_ref, sup_rgb_ref, sup_flow_ref,
                   out_ref_rgb_ref, out_sup_rgb_ref, m_intra_ref, m_inter_ref,
                   e_ref):
    rn = _normalize(ref_flow_ref[0])
    sn = _normalize(sup_flow_ref[0])

    for t in range(_N // _IT):
        lo, hi = t * _IT, (t + 1) * _IT
        st_intra = jax.lax.dot_general(
            rn, rn[lo:hi, :], (((1,), (1,)), ((), ())),
            preferred_element_type=jnp.float32)
        m_intra_ref[0:1, lo:hi] = _softmax_mean_cols(st_intra, e_ref)
        st_inter = jax.lax.dot_general(
            rn, sn[lo:hi, :], (((1,), (1,)), ((), ())),
            preferred_element_type=jnp.float32)
        m_inter_ref[0:1, lo:hi] = _softmax_mean_cols(st_inter, e_ref)

    iota = jax.lax.broadcasted_iota(jnp.int32, (1, _N), 1)

    m = m_intra_ref[0, :][None, :]
    for k in range(_K):
        v = jnp.min(m)
        idx = jnp.min(jnp.where(m == v, iota, _N))
        out_ref_rgb_ref[0, k, :] = ref_rgb_ref[0, pl.ds(idx, 1), :][0]
        m = jnp.where(iota == idx, jnp.inf, m)

    m = m_inter_ref[0, :][None, :]
    for k in range(_K):
        v = jnp.max(m)
        idx = jnp.min(jnp.where(m == v, iota, _N))
        out_sup_rgb_ref[0, k, :] = sup_rgb_ref[0, pl.ds(idx, 1), :][0]
        m = jnp.where(iota == idx, -jnp.inf, m)


def _run(ref_rgb_feat, ref_flow_feat, sup_rgb_feat, sup_flow_feat):
    B = ref_rgb_feat.shape[0]
    feat_spec = pl.BlockSpec((1, _N, _F), lambda b: (b, 0, 0))
    out_spec = pl.BlockSpec((1, _K, _F), lambda b: (b, 0, 0))
    out_rgb, out_sup = pl.pallas_call(
        _fusion_kernel,
        grid=(B,),
        in_specs=[feat_spec, feat_spec, feat_spec, feat_spec],
        out_specs=[out_spec, out_spec],
        out_shape=[
            jax.ShapeDtypeStruct((B, _K, _F), jnp.float32),
            jax.ShapeDtypeStruct((B, _K, _F), jnp.float32),
        ],
        scratch_shapes=[
            pltpu.VMEM((1, _N), jnp.float32),
            pltpu.VMEM((1, _N), jnp.float32),
            pltpu.VMEM((_N, _IT), jnp.float32),
        ],
    )(ref_rgb_feat, ref_flow_feat, sup_rgb_feat, sup_flow_feat)
    return (out_rgb, out_sup)


@jax.jit
def kernel(ref_rgb_feat, ref_flow_feat, sup_rgb_feat, sup_flow_feat):
    return _run(ref_rgb_feat, ref_flow_feat, sup_rgb_feat, sup_flow_feat)


# speedup vs baseline: 4.1382x; 2.5638x over previous
import jax
import jax.numpy as jnp
from jax.experimental import pallas as pl
from jax.experimental.pallas import tpu as pltpu

_N = 2048
_F = 32
_K = 8
_IT = 512


def _colsum(x):
    acc = jnp.sum(x.reshape(_N // 8, 8, x.shape[-1]), axis=0)
    return _halve8(acc)


def _normalize(x):
    sq = x * x
    acc = sq[:, 0:8] + sq[:, 8:16]
    acc = acc + sq[:, 16:24]
    acc = acc + sq[:, 24:32]
    r4 = acc[:, 0:4] + acc[:, 4:8]
    r2 = r4[:, 0:2] + r4[:, 2:4]
    nsq = r2[:, 0:1] + r2[:, 1:2]
    return x / jnp.sqrt(nsq)


def _halve8(acc):
    r4 = acc[:4, :] + acc[4:, :]
    r2 = r4[:2, :] + r4[2:, :]
    return r2[0:1, :] + r2[1:2, :]


def _softmax_mean_cols(st, e_ref):
    del e_ref
    cmax = jnp.max(st, axis=0, keepdims=True)
    e = jnp.exp(st - cmax)
    den = _colsum(e)
    return _colsum(e / den) * jnp.float32(1.0 / _N)


def _fusion_kernel(ref_rgb_ref, ref_flow_ref, sup_rgb_ref, sup_flow_ref,
                   out_ref_rgb_ref, out_sup_rgb_ref, m_intra_ref, m_inter_ref,
                   e_ref):
    rn = _normalize(ref_flow_ref[0])
    sn = _normalize(sup_flow_ref[0])

    for t in range(_N // _IT):
        lo, hi = t * _IT, (t + 1) * _IT
        st_intra = jax.lax.dot_general(
            rn, rn[lo:hi, :], (((1,), (1,)), ((), ())),
            preferred_element_type=jnp.float32)
        m_intra_ref[0:1, lo:hi] = _softmax_mean_cols(st_intra, e_ref)
        st_inter = jax.lax.dot_general(
            rn, sn[lo:hi, :], (((1,), (1,)), ((), ())),
            preferred_element_type=jnp.float32)
        m_inter_ref[0:1, lo:hi] = _softmax_mean_cols(st_inter, e_ref)

    iota = jax.lax.broadcasted_iota(jnp.int32, (1, _N), 1)

    m = m_intra_ref[0, :][None, :]
    for k in range(_K):
        v = jnp.min(m)
        idx = jnp.min(jnp.where(m == v, iota, _N))
        out_ref_rgb_ref[0, k, :] = ref_rgb_ref[0, pl.ds(idx, 1), :][0]
        m = jnp.where(iota == idx, jnp.inf, m)

    m = m_inter_ref[0, :][None, :]
    for k in range(_K):
        v = jnp.max(m)
        idx = jnp.min(jnp.where(m == v, iota, _N))
        out_sup_rgb_ref[0, k, :] = sup_rgb_ref[0, pl.ds(idx, 1), :][0]
        m = jnp.where(iota == idx, -jnp.inf, m)


def _run(ref_rgb_feat, ref_flow_feat, sup_rgb_feat, sup_flow_feat):
    B = ref_rgb_feat.shape[0]
    feat_spec = pl.BlockSpec((1, _N, _F), lambda b: (b, 0, 0))
    out_spec = pl.BlockSpec((1, _K, _F), lambda b: (b, 0, 0))
    out_rgb, out_sup = pl.pallas_call(
        _fusion_kernel,
        grid=(B,),
        in_specs=[feat_spec, feat_spec, feat_spec, feat_spec],
        out_specs=[out_spec, out_spec],
        out_shape=[
            jax.ShapeDtypeStruct((B, _K, _F), jnp.float32),
            jax.ShapeDtypeStruct((B, _K, _F), jnp.float32),
        ],
        scratch_shapes=[
            pltpu.VMEM((1, _N), jnp.float32),
            pltpu.VMEM((1, _N), jnp.float32),
            pltpu.VMEM((_N, _IT), jnp.float32),
        ],
    )(ref_rgb_feat, ref_flow_feat, sup_rgb_feat, sup_flow_feat)
    return (out_rgb, out_sup)


@jax.jit
def kernel(ref_rgb_feat, ref_flow_feat, sup_rgb_feat, sup_flow_feat):
    return _run(ref_rgb_feat, ref_flow_feat, sup_rgb_feat, sup_flow_feat)


# speedup vs baseline: 4.6494x; 1.1235x over previous
import jax
import jax.numpy as jnp
from jax.experimental import pallas as pl
from jax.experimental.pallas import tpu as pltpu

_N = 2048
_F = 32
_K = 8
_IT = 1024


def _colsum(x):
    acc = jnp.sum(x.reshape(_N // 8, 8, x.shape[-1]), axis=0)
    return _halve8(acc)


def _normalize(x):
    sq = x * x
    acc = sq[:, 0:8] + sq[:, 8:16]
    acc = acc + sq[:, 16:24]
    acc = acc + sq[:, 24:32]
    r4 = acc[:, 0:4] + acc[:, 4:8]
    r2 = r4[:, 0:2] + r4[:, 2:4]
    nsq = r2[:, 0:1] + r2[:, 1:2]
    return x / jnp.sqrt(nsq)


def _halve8(acc):
    r4 = acc[:4, :] + acc[4:, :]
    r2 = r4[:2, :] + r4[2:, :]
    return r2[0:1, :] + r2[1:2, :]


def _softmax_mean_cols(st, e_ref):
    del e_ref
    cmax = jnp.max(st, axis=0, keepdims=True)
    dacc = jnp.exp(st[0:8, :] - cmax)
    for c in range(1, _N // 8):
        dacc = dacc + jnp.exp(st[8 * c:8 * (c + 1), :] - cmax)
    den = _halve8(dacc)
    acc = jnp.exp(st[0:8, :] - cmax) / den
    for c in range(1, _N // 8):
        acc = acc + jnp.exp(st[8 * c:8 * (c + 1), :] - cmax) / den
    return _halve8(acc) * jnp.float32(1.0 / _N)


def _fusion_kernel(ref_rgb_ref, ref_flow_ref, sup_rgb_ref, sup_flow_ref,
                   out_ref_rgb_ref, out_sup_rgb_ref, m_intra_ref, m_inter_ref,
                   e_ref):
    rn = _normalize(ref_flow_ref[0])
    sn = _normalize(sup_flow_ref[0])

    for t in range(_N // _IT):
        lo, hi = t * _IT, (t + 1) * _IT
        st_intra = jax.lax.dot_general(
            rn, rn[lo:hi, :], (((1,), (1,)), ((), ())),
            preferred_element_type=jnp.float32)
        st_inter = jax.lax.dot_general(
            rn, sn[lo:hi, :], (((1,), (1,)), ((), ())),
            preferred_element_type=jnp.float32)
        m_intra_ref[0:1, lo:hi] = _softmax_mean_cols(st_intra, e_ref)
        m_inter_ref[0:1, lo:hi] = _softmax_mean_cols(st_inter, e_ref)

    iota = jax.lax.broadcasted_iota(jnp.int32, (1, _N), 1)

    m = m_intra_ref[0, :][None, :]
    for k in range(_K):
        v = jnp.min(m)
        idx = jnp.min(jnp.where(m == v, iota, _N))
        out_ref_rgb_ref[0, k, :] = ref_rgb_ref[0, pl.ds(idx, 1), :][0]
        m = jnp.where(iota == idx, jnp.inf, m)

    m = m_inter_ref[0, :][None, :]
    for k in range(_K):
        v = jnp.max(m)
        idx = jnp.min(jnp.where(m == v, iota, _N))
        out_sup_rgb_ref[0, k, :] = sup_rgb_ref[0, pl.ds(idx, 1), :][0]
        m = jnp.where(iota == idx, -jnp.inf, m)


def _run(ref_rgb_feat, ref_flow_feat, sup_rgb_feat, sup_flow_feat):
    B = ref_rgb_feat.shape[0]
    feat_spec = pl.BlockSpec((1, _N, _F), lambda b: (b, 0, 0))
    out_spec = pl.BlockSpec((1, _K, _F), lambda b: (b, 0, 0))
    out_rgb, out_sup = pl.pallas_call(
        _fusion_kernel,
        grid=(B,),
        in_specs=[feat_spec, feat_spec, feat_spec, feat_spec],
        out_specs=[out_spec, out_spec],
        out_shape=[
            jax.ShapeDtypeStruct((B, _K, _F), jnp.float32),
            jax.ShapeDtypeStruct((B, _K, _F), jnp.float32),
        ],
        scratch_shapes=[
            pltpu.VMEM((1, _N), jnp.float32),
            pltpu.VMEM((1, _N), jnp.float32),
            pltpu.VMEM((_N, _IT), jnp.float32),
        ],
    )(ref_rgb_feat, ref_flow_feat, sup_rgb_feat, sup_flow_feat)
    return (out_rgb, out_sup)


@jax.jit
def kernel(ref_rgb_feat, ref_flow_feat, sup_rgb_feat, sup_flow_feat):
    return _run(ref_rgb_feat, ref_flow_feat, sup_rgb_feat, sup_flow_feat)


# speedup vs baseline: 4.7546x; 1.0226x over previous
import jax
import jax.numpy as jnp
from jax.experimental import pallas as pl
from jax.experimental.pallas import tpu as pltpu

_N = 2048
_F = 32
_K = 8
_IT = 2048
_BPS = 1


def _colsum(x):
    acc = jnp.sum(x.reshape(_N // 8, 8, x.shape[-1]), axis=0)
    return _halve8(acc)


def _normalize(x):
    sq = x * x
    acc = sq[:, 0:8] + sq[:, 8:16]
    acc = acc + sq[:, 16:24]
    acc = acc + sq[:, 24:32]
    r4 = acc[:, 0:4] + acc[:, 4:8]
    r2 = r4[:, 0:2] + r4[:, 2:4]
    nsq = r2[:, 0:1] + r2[:, 1:2]
    return x / jnp.sqrt(nsq)


def _halve8(acc):
    r4 = acc[:4, :] + acc[4:, :]
    r2 = r4[:2, :] + r4[2:, :]
    return r2[0:1, :] + r2[1:2, :]


def _softmax_mean_cols(st):
    cmax = jnp.max(st, axis=0, keepdims=True)
    dacc = jnp.exp(st[0:8, :] - cmax)
    for c in range(1, _N // 8):
        dacc = dacc + jnp.exp(st[8 * c:8 * (c + 1), :] - cmax)
    den = _halve8(dacc)
    acc = jnp.exp(st[0:8, :] - cmax) / den
    for c in range(1, _N // 8):
        acc = acc + jnp.exp(st[8 * c:8 * (c + 1), :] - cmax) / den
    return _halve8(acc) * jnp.float32(1.0 / _N)


def _fusion_kernel(ref_rgb_ref, ref_flow_ref, sup_rgb_ref, sup_flow_ref,
                   out_ref_rgb_ref, out_sup_rgb_ref, m_intra_ref, m_inter_ref):
    for bb in range(_BPS):
        rn = _normalize(ref_flow_ref[bb])
        sn = _normalize(sup_flow_ref[bb])

        for t in range(_N // _IT):
            lo, hi = t * _IT, (t + 1) * _IT
            st_intra = jax.lax.dot_general(
                rn, rn[lo:hi, :], (((1,), (1,)), ((), ())),
                preferred_element_type=jnp.float32)
            st_inter = jax.lax.dot_general(
                rn, sn[lo:hi, :], (((1,), (1,)), ((), ())),
                preferred_element_type=jnp.float32)
            m_intra_ref[bb:bb + 1, lo:hi] = _softmax_mean_cols(st_intra)
            m_inter_ref[bb:bb + 1, lo:hi] = _softmax_mean_cols(st_inter)

    iota = jax.lax.broadcasted_iota(jnp.int32, (1, _N), 1)

    for bb in range(_BPS):
        m = m_intra_ref[bb, :][None, :]
        for k in range(_K):
            v = jnp.min(m)
            idx = jnp.min(jnp.where(m == v, iota, _N))
            out_ref_rgb_ref[bb, k, :] = ref_rgb_ref[bb, pl.ds(idx, 1), :][0]
            m = jnp.where(iota == idx, jnp.inf, m)

        m = m_inter_ref[bb, :][None, :]
        for k in range(_K):
            v = jnp.max(m)
            idx = jnp.min(jnp.where(m == v, iota, _N))
            out_sup_rgb_ref[bb, k, :] = sup_rgb_ref[bb, pl.ds(idx, 1), :][0]
            m = jnp.where(iota == idx, -jnp.inf, m)


def _run(ref_rgb_feat, ref_flow_feat, sup_rgb_feat, sup_flow_feat):
    B = ref_rgb_feat.shape[0]
    feat_spec = pl.BlockSpec((_BPS, _N, _F), lambda b: (b, 0, 0))
    out_spec = pl.BlockSpec((_BPS, _K, _F), lambda b: (b, 0, 0))
    out_rgb, out_sup = pl.pallas_call(
        _fusion_kernel,
        grid=(B // _BPS,),
        in_specs=[feat_spec, feat_spec, feat_spec, feat_spec],
        out_specs=[out_spec, out_spec],
        out_shape=[
            jax.ShapeDtypeStruct((B, _K, _F), jnp.float32),
            jax.ShapeDtypeStruct((B, _K, _F), jnp.float32),
        ],
        scratch_shapes=[
            pltpu.VMEM((_BPS, _N), jnp.float32),
            pltpu.VMEM((_BPS, _N), jnp.float32),
        ],
        compiler_params=pltpu.CompilerParams(
            dimension_semantics=("parallel",)),
    )(ref_rgb_feat, ref_flow_feat, sup_rgb_feat, sup_flow_feat)
    return (out_rgb, out_sup)


@jax.jit
def kernel(ref_rgb_feat, ref_flow_feat, sup_rgb_feat, sup_flow_feat):
    return _run(ref_rgb_feat, ref_flow_feat, sup_rgb_feat, sup_flow_feat)


# speedup vs baseline: 5.2084x; 1.0954x over previous
import jax
import jax.numpy as jnp
from jax.experimental import pallas as pl
from jax.experimental.pallas import tpu as pltpu

_N = 2048
_F = 32
_K = 8
_IT = 2048
_BPS = 1


def _colsum(x):
    acc = jnp.sum(x.reshape(_N // 8, 8, x.shape[-1]), axis=0)
    return _halve8(acc)


def _normalize(x):
    xT = jnp.transpose(x)
    sq = xT * xT
    acc = sq[0:8, :] + sq[8:16, :]
    acc = acc + sq[16:24, :]
    acc = acc + sq[24:32, :]
    r4 = acc[0:4, :] + acc[4:8, :]
    r2 = r4[0:2, :] + r4[2:4, :]
    nsq = r2[0:1, :] + r2[1:2, :]
    return jnp.transpose(xT / jnp.sqrt(nsq))


def _halve8(acc):
    r4 = acc[:4, :] + acc[4:, :]
    r2 = r4[:2, :] + r4[2:, :]
    return r2[0:1, :] + r2[1:2, :]


def _softmax_mean_cols(st):
    cmax = jnp.max(st, axis=0, keepdims=True)
    dacc = jnp.exp(st[0:8, :] - cmax)
    for c in range(1, _N // 8):
        dacc = dacc + jnp.exp(st[8 * c:8 * (c + 1), :] - cmax)
    den = _halve8(dacc)
    acc = jnp.exp(st[0:8, :] - cmax) / den
    for c in range(1, _N // 8):
        acc = acc + jnp.exp(st[8 * c:8 * (c + 1), :] - cmax) / den
    return _halve8(acc) * jnp.float32(1.0 / _N)


def _fusion_kernel(ref_rgb_ref, ref_flow_ref, sup_rgb_ref, sup_flow_ref,
                   out_ref_rgb_ref, out_sup_rgb_ref, m_intra_ref, m_inter_ref):
    for bb in range(_BPS):
        rn = _normalize(ref_flow_ref[bb])
        sn = _normalize(sup_flow_ref[bb])

        for t in range(_N // _IT):
            lo, hi = t * _IT, (t + 1) * _IT
            st_intra = jax.lax.dot_general(
                rn, rn[lo:hi, :], (((1,), (1,)), ((), ())),
                preferred_element_type=jnp.float32)
            st_inter = jax.lax.dot_general(
                rn, sn[lo:hi, :], (((1,), (1,)), ((), ())),
                preferred_element_type=jnp.float32)
            m_intra_ref[bb:bb + 1, lo:hi] = _softmax_mean_cols(st_intra)
            m_inter_ref[bb:bb + 1, lo:hi] = _softmax_mean_cols(st_inter)

    iota = jax.lax.broadcasted_iota(jnp.int32, (1, _N), 1)

    for bb in range(_BPS):
        m = m_intra_ref[bb, :][None, :]
        for k in range(_K):
            v = jnp.min(m)
            idx = jnp.min(jnp.where(m == v, iota, _N))
            out_ref_rgb_ref[bb, k, :] = ref_rgb_ref[bb, pl.ds(idx, 1), :][0]
            m = jnp.where(iota == idx, jnp.inf, m)

        m = m_inter_ref[bb, :][None, :]
        for k in range(_K):
            v = jnp.max(m)
            idx = jnp.min(jnp.where(m == v, iota, _N))
            out_sup_rgb_ref[bb, k, :] = sup_rgb_ref[bb, pl.ds(idx, 1), :][0]
            m = jnp.where(iota == idx, -jnp.inf, m)


def _run(ref_rgb_feat, ref_flow_feat, sup_rgb_feat, sup_flow_feat):
    B = ref_rgb_feat.shape[0]
    feat_spec = pl.BlockSpec((_BPS, _N, _F), lambda b: (b, 0, 0))
    out_spec = pl.BlockSpec((_BPS, _K, _F), lambda b: (b, 0, 0))
    out_rgb, out_sup = pl.pallas_call(
        _fusion_kernel,
        grid=(B // _BPS,),
        in_specs=[feat_spec, feat_spec, feat_spec, feat_spec],
        out_specs=[out_spec, out_spec],
        out_shape=[
            jax.ShapeDtypeStruct((B, _K, _F), jnp.float32),
            jax.ShapeDtypeStruct((B, _K, _F), jnp.float32),
        ],
        scratch_shapes=[
            pltpu.VMEM((_BPS, _N), jnp.float32),
            pltpu.VMEM((_BPS, _N), jnp.float32),
        ],
        compiler_params=pltpu.CompilerParams(
            dimension_semantics=("parallel",)),
    )(ref_rgb_feat, ref_flow_feat, sup_rgb_feat, sup_flow_feat)
    return (out_rgb, out_sup)


@jax.jit
def kernel(ref_rgb_feat, ref_flow_feat, sup_rgb_feat, sup_flow_feat):
    return _run(ref_rgb_feat, ref_flow_feat, sup_rgb_feat, sup_flow_feat)


# speedup vs baseline: 5.2117x; 1.0006x over previous
import jax
import jax.numpy as jnp
from jax.experimental import pallas as pl
from jax.experimental.pallas import tpu as pltpu

_N = 2048
_F = 32
_K = 8
_IT = 2048
_BPS = 1


def _normalize(x):
    xT = jnp.transpose(x)
    sq = xT * xT
    acc = sq[0:8, :] + sq[8:16, :]
    acc = acc + sq[16:24, :]
    acc = acc + sq[24:32, :]
    r4 = acc[0:4, :] + acc[4:8, :]
    r2 = r4[0:2, :] + r4[2:4, :]
    nsq = r2[0:1, :] + r2[1:2, :]
    return jnp.transpose(xT / jnp.sqrt(nsq))


def _halve8(acc):
    r4 = acc[:4, :] + acc[4:, :]
    r2 = r4[:2, :] + r4[2:, :]
    return r2[0:1, :] + r2[1:2, :]


def _softmax_mean_cols(st):
    cmax = jnp.max(st, axis=0, keepdims=True)
    dacc = jnp.exp(st[0:8, :] - cmax)
    for c in range(1, _N // 8):
        dacc = dacc + jnp.exp(st[8 * c:8 * (c + 1), :] - cmax)
    den = _halve8(dacc)
    acc = jnp.exp(st[0:8, :] - cmax) / den
    for c in range(1, _N // 8):
        acc = acc + jnp.exp(st[8 * c:8 * (c + 1), :] - cmax) / den
    return _halve8(acc) * jnp.float32(1.0 / _N)


def _fusion_kernel(ref_rgb_ref, ref_flow_ref, sup_rgb_ref, sup_flow_ref,
                   out_ref_rgb_ref, out_sup_rgb_ref, m_intra_ref, m_inter_ref):
    for bb in range(_BPS):
        rn = _normalize(ref_flow_ref[bb])
        sn = _normalize(sup_flow_ref[bb])

        for t in range(_N // _IT):
            lo, hi = t * _IT, (t + 1) * _IT
            st_intra = jax.lax.dot_general(
                rn, rn[lo:hi, :], (((1,), (1,)), ((), ())),
                preferred_element_type=jnp.float32)
            st_inter = jax.lax.dot_general(
                rn, sn[lo:hi, :], (((1,), (1,)), ((), ())),
                preferred_element_type=jnp.float32)
            m_intra_ref[bb:bb + 1, lo:hi] = _softmax_mean_cols(st_intra)
            m_inter_ref[bb:bb + 1, lo:hi] = _softmax_mean_cols(st_inter)

    iota = jax.lax.broadcasted_iota(jnp.int32, (1, _N), 1)

    for bb in range(_BPS):
        m = m_intra_ref[bb, :][None, :]
        for k in range(_K):
            v = jnp.min(m)
            idx = jnp.min(jnp.where(m == v, iota, _N))
            out_ref_rgb_ref[bb, k, :] = ref_rgb_ref[bb, pl.ds(idx, 1), :][0]
            m = jnp.where(iota == idx, jnp.inf, m)

        m = m_inter_ref[bb, :][None, :]
        for k in range(_K):
            v = jnp.max(m)
            idx = jnp.min(jnp.where(m == v, iota, _N))
            out_sup_rgb_ref[bb, k, :] = sup_rgb_ref[bb, pl.ds(idx, 1), :][0]
            m = jnp.where(iota == idx, -jnp.inf, m)


def _run(ref_rgb_feat, ref_flow_feat, sup_rgb_feat, sup_flow_feat):
    B = ref_rgb_feat.shape[0]
    feat_spec = pl.BlockSpec((_BPS, _N, _F), lambda b: (b, 0, 0))
    out_spec = pl.BlockSpec((_BPS, _K, _F), lambda b: (b, 0, 0))
    out_rgb, out_sup = pl.pallas_call(
        _fusion_kernel,
        grid=(B // _BPS,),
        in_specs=[feat_spec, feat_spec, feat_spec, feat_spec],
        out_specs=[out_spec, out_spec],
        out_shape=[
            jax.ShapeDtypeStruct((B, _K, _F), jnp.float32),
            jax.ShapeDtypeStruct((B, _K, _F), jnp.float32),
        ],
        scratch_shapes=[
            pltpu.VMEM((_BPS, _N), jnp.float32),
            pltpu.VMEM((_BPS, _N), jnp.float32),
        ],
        compiler_params=pltpu.CompilerParams(
            dimension_semantics=("parallel",)),
    )(ref_rgb_feat, ref_flow_feat, sup_rgb_feat, sup_flow_feat)
    return (out_rgb, out_sup)


@jax.jit
def kernel(ref_rgb_feat, ref_flow_feat, sup_rgb_feat, sup_flow_feat):
    return _run(ref_rgb_feat, ref_flow_feat, sup_rgb_feat, sup_flow_feat)
